# baseline (speedup 1.0000x reference)
import functools
import jax
import jax.numpy as jnp
from jax import lax
from jax.experimental import pallas as pl
from jax.experimental.pallas import tpu as pltpu
from jax.experimental.pallas import tpu_sc as plsc

_E, _TOPK, _H, _I = 16, 2, 1024, 512
_NORM_MIN = 1e-12
_TB = 256
_T = 2048
_A = _T * _TOPK
_RB = 128
_NBLK = _A // _RB + _E
_PAD_ROWS = _NBLK * _RB
_DUMP = _A
_CAP = _T + _RB
_L = 16


def _router_kernel(x_ref, gwt_ref, logits_ref, w_ref, sel_ref, hist_ref):
    x = x_ref[...]
    logits = jnp.dot(x, gwt_ref[...], preferred_element_type=jnp.float32)
    logits_ref[...] = logits
    m = jnp.max(logits, axis=-1, keepdims=True)
    ex = jnp.exp(logits - m)
    probs = ex / jnp.sum(ex, axis=-1, keepdims=True)
    iota = lax.broadcasted_iota(jnp.int32, probs.shape, 1)
    m0 = jnp.max(probs, axis=-1, keepdims=True)
    e0 = jnp.min(jnp.where(probs == m0, iota, _E), axis=-1, keepdims=True)
    probs1 = jnp.where(iota == e0, -1.0, probs)
    m1 = jnp.max(probs1, axis=-1, keepdims=True)
    e1 = jnp.min(jnp.where(probs1 == m1, iota, _E), axis=-1, keepdims=True)
    s = jnp.maximum(m0 + m1, _NORM_MIN)
    w_ref[...] = jnp.concatenate([m0 / s, m1 / s], axis=1)
    sel_ref[...] = jnp.concatenate([e0, e1], axis=1)
    onehot = (iota == e0).astype(jnp.int32) + (iota == e1).astype(jnp.int32)
    hist_ref[...] = jnp.sum(onehot, axis=0, keepdims=True)[None]


def _splat(x):
    return lax.broadcast_in_dim(jnp.asarray(x, jnp.int32), (_L,), ())


def _dispatch_body(eids_hbm, x_hbm, hist_hbm, xs_hbm, ra_hbm, b2e_hbm,
                   eids_v, ra_buf, tok_buf, rows_v, b2e_v, hist_v,
                   dump_v, sem):
    cid = lax.axis_index("c")
    sid = lax.axis_index("s")
    lanes = lax.iota(jnp.int32, _L)
    sid_v = _splat(sid)
    zero_v = jnp.zeros((_L,), jnp.int32)
    one_v = jnp.ones((_L,), jnp.int32)
    dump_c = jnp.full((_L,), _DUMP, jnp.int32)

    pltpu.sync_copy(eids_hbm, eids_v)

    for k in range(_CAP // _L):
        ra_buf[pl.ds(k * _L, _L)] = dump_c
    for k in range(_RB // _L):
        dump_v[pl.ds(k * _L, _L)] = dump_c

    def cbody(j, cnt_splat):
        v = eids_v[pl.ds(j * _L, _L)]
        msk = v == sid_v
        idxvec = lanes + _splat(j * _L)
        rank = cnt_splat + jnp.cumsum(jnp.where(msk, one_v, zero_v)) - one_v
        rank = jnp.where(msk, rank, zero_v)
        plsc.store_scatter(ra_buf, [rank], idxvec, mask=msk)
        return cnt_splat + plsc.all_reduce_population_count(msk)

    cnt_splat = lax.fori_loop(0, _A // _L, cbody, jnp.zeros((_L,), jnp.int32))

    pltpu.sync_copy(hist_hbm, hist_v)
    counts_vec = jnp.zeros((_L,), jnp.int32)
    for r in range(_T // _TB):
        counts_vec = counts_vec + hist_v[r]

    g_vec = lax.shift_right_logical(counts_vec + jnp.full((_L,), _RB - 1, jnp.int32),
                                    jnp.full((_L,), 7, jnp.int32))
    cum_incl = jnp.cumsum(g_vec)
    pstart_vec = (cum_incl - g_vec) * jnp.full((_L,), _RB, jnp.int32)
    mymask = lanes == sid_v
    g_s = jnp.sum(jnp.where(mymask, g_vec, zero_v))
    pstart = jnp.sum(jnp.where(mymask, pstart_vec, zero_v))
    total_blocks = jnp.sum(jnp.where(lanes == jnp.full((_L,), _L - 1, jnp.int32),
                                     cum_incl, zero_v))

    for k in range(_CAP // _L):
        v = ra_buf[pl.ds(k * _L, _L)]
        tok_buf[pl.ds(k * _L, _L)] = jnp.where(v == dump_c, zero_v,
                                               lax.shift_right_logical(v, one_v))

    @pl.when(cid == 0)
    def _():
        def wr(k, carry):
            off = pl.multiple_of(k * _RB, _RB)
            pltpu.sync_copy(ra_buf.at[pl.ds(off, _RB)],
                            ra_hbm.at[pl.ds(pl.multiple_of(pstart + off, _RB), _RB)])
            return carry
        lax.fori_loop(0, g_s, wr, 0)

    @pl.when((cid == 1) & (sid == 0))
    def _():
        def wrtail(k, carry):
            pltpu.sync_copy(dump_v, ra_hbm.at[pl.ds(pl.multiple_of(k * _RB, _RB), _RB)])
            return carry
        lax.fori_loop(total_blocks, _NBLK, wrtail, 0)

    @pl.when((cid == 0) & (sid == 0))
    def _():
        for jj in range(_NBLK // _L):
            bvec = lanes + jnp.full((_L,), jj * _L, jnp.int32)
            acc = jnp.zeros((_L,), jnp.int32)
            for e in range(_E):
                ce_v = jnp.where(lanes == jnp.full((_L,), e, jnp.int32),
                                 cum_incl, zero_v)
                ce_v = _splat(jnp.sum(ce_v))
                acc += jnp.where(bvec >= ce_v, one_v, zero_v)
            b2e_v[pl.ds(jj * _L, _L)] = jnp.minimum(acc, jnp.full((_L,), _E - 1, jnp.int32))
        pltpu.sync_copy(b2e_v, b2e_hbm)

    def gbody(i, carry):
        m = cid + 2 * i
        off = pl.multiple_of(m * 64, 64)
        idx = tok_buf.at[pl.ds(off, 64)]
        pltpu.async_copy(x_hbm.at[idx], rows_v, sem).wait()
        pltpu.sync_copy(rows_v, xs_hbm.at[pl.ds(pl.multiple_of(pstart + off, 64), 64)])
        return carry

    lax.fori_loop(0, g_s, gbody, 0)


def _gmm_kernel(b2e_ref, x_ref, gw_ref, uw_ref, dw_ref, out_ref):
    x = x_ref[...]
    g = jnp.dot(x, gw_ref[0], preferred_element_type=jnp.float32)
    u = jnp.dot(x, uw_ref[0], preferred_element_type=jnp.float32)
    h = g * jax.nn.sigmoid(g) * u
    out_ref[...] = jnp.dot(h, dw_ref[0], preferred_element_type=jnp.float32)


def _unsort_body(ra_hbm, ys_hbm, ytok_hbm, idx_v, rows_v, sem):
    cid = lax.axis_index("c")
    sid = lax.axis_index("s")
    w = sid * 2 + cid
    rows_per_w = _PAD_ROWS // 32
    for j in range(rows_per_w // 64):
        base = pl.multiple_of(w * rows_per_w + j * 64, 64)
        pltpu.sync_copy(ra_hbm.at[pl.ds(base, 64)], idx_v)
        pltpu.sync_copy(ys_hbm.at[pl.ds(base, 64)], rows_v)
        pltpu.sync_copy(rows_v, ytok_hbm.at[idx_v])


def _shared_combine_kernel(x_ref, w_ref, yt_ref, sg_ref, su_ref, sd_ref, out_ref):
    x = x_ref[...]
    g = jnp.dot(x, sg_ref[...], preferred_element_type=jnp.float32)
    u = jnp.dot(x, su_ref[...], preferred_element_type=jnp.float32)
    h = g * jax.nn.sigmoid(g) * u
    sh = jnp.dot(h, sd_ref[...], preferred_element_type=jnp.float32)
    yt = yt_ref[...]
    w = w_ref[...]
    out_ref[...] = sh + yt[:, 0, :] * w[:, 0:1] + yt[:, 1, :] * w[:, 1:2]


def kernel(hidden_states, gate_w, expert_gate_w, expert_up_w, expert_down_w,
           shared_gate_w, shared_up_w, shared_down_w):
    b, s, hd = hidden_states.shape
    x = hidden_states.reshape(-1, hd)
    T = x.shape[0]
    nb = T // _TB

    logits, w, sel, hist = pl.pallas_call(
        _router_kernel,
        grid=(nb,),
        in_specs=[
            pl.BlockSpec((_TB, _H), lambda i: (i, 0)),
            pl.BlockSpec((_H, _E), lambda i: (0, 0)),
        ],
        out_specs=[
            pl.BlockSpec((_TB, _E), lambda i: (i, 0)),
            pl.BlockSpec((_TB, _TOPK), lambda i: (i, 0)),
            pl.BlockSpec((_TB, _TOPK), lambda i: (i, 0)),
            pl.BlockSpec((1, 1, _E), lambda i: (i, 0, 0)),
        ],
        out_shape=[
            jax.ShapeDtypeStruct((T, _E), jnp.float32),
            jax.ShapeDtypeStruct((T, _TOPK), jnp.float32),
            jax.ShapeDtypeStruct((T, _TOPK), jnp.int32),
            jax.ShapeDtypeStruct((T // _TB, 1, _E), jnp.int32),
        ],
    )(x, gate_w.T)

    eids = sel.reshape(-1)

    mesh = plsc.VectorSubcoreMesh(core_axis_name="c", subcore_axis_name="s",
                                  num_cores=2)
    dispatch = pl.kernel(
        _dispatch_body,
        out_type=[
            jax.ShapeDtypeStruct((_PAD_ROWS, _H), jnp.float32),
            jax.ShapeDtypeStruct((_PAD_ROWS,), jnp.int32),
            jax.ShapeDtypeStruct((_NBLK,), jnp.int32),
        ],
        mesh=mesh,
        compiler_params=pltpu.CompilerParams(needs_layout_passes=False),
        scratch_types=[
            pltpu.VMEM((_A,), jnp.int32),
            pltpu.VMEM((_CAP,), jnp.int32),
            pltpu.VMEM((_CAP,), jnp.int32),
            pltpu.VMEM((64, _H), jnp.float32),
            pltpu.VMEM((_NBLK,), jnp.int32),
            pltpu.VMEM((_T // _TB, _L), jnp.int32),
            pltpu.VMEM((_RB,), jnp.int32),
            pltpu.SemaphoreType.DMA,
        ],
    )
    xs, ra, b2e = dispatch(eids, x, hist.reshape(T // _TB, _E))

    ys = pl.pallas_call(
        _gmm_kernel,
        grid_spec=pltpu.PrefetchScalarGridSpec(
            num_scalar_prefetch=1,
            grid=(_NBLK,),
            in_specs=[
                pl.BlockSpec((_RB, _H), lambda j, b2e_s: (j, 0)),
                pl.BlockSpec((1, _H, _I), lambda j, b2e_s: (b2e_s[j], 0, 0)),
                pl.BlockSpec((1, _H, _I), lambda j, b2e_s: (b2e_s[j], 0, 0)),
                pl.BlockSpec((1, _I, _H), lambda j, b2e_s: (b2e_s[j], 0, 0)),
            ],
            out_specs=pl.BlockSpec((_RB, _H), lambda j, b2e_s: (j, 0)),
        ),
        out_shape=jax.ShapeDtypeStruct((_PAD_ROWS, _H), jnp.float32),
    )(b2e, xs, expert_gate_w, expert_up_w, expert_down_w)

    unsort = pl.kernel(
        _unsort_body,
        out_type=jax.ShapeDtypeStruct((_A + 1, _H), jnp.float32),
        mesh=plsc.VectorSubcoreMesh(core_axis_name="c", subcore_axis_name="s",
                                    num_cores=2),
        compiler_params=pltpu.CompilerParams(needs_layout_passes=False),
        scratch_types=[
            pltpu.VMEM((64,), jnp.int32),
            pltpu.VMEM((64, _H), jnp.float32),
            pltpu.SemaphoreType.DMA,
        ],
    )
    ytok = unsort(ra, ys)
    yt3 = ytok[:_A].reshape(T, _TOPK, _H)

    out = pl.pallas_call(
        _shared_combine_kernel,
        grid=(nb,),
        in_specs=[
            pl.BlockSpec((_TB, _H), lambda i: (i, 0)),
            pl.BlockSpec((_TB, _TOPK), lambda i: (i, 0)),
            pl.BlockSpec((_TB, _TOPK, _H), lambda i: (i, 0, 0)),
            pl.BlockSpec((_H, _I), lambda i: (0, 0)),
            pl.BlockSpec((_H, _I), lambda i: (0, 0)),
            pl.BlockSpec((_I, _H), lambda i: (0, 0)),
        ],
        out_specs=pl.BlockSpec((_TB, _H), lambda i: (i, 0)),
        out_shape=jax.ShapeDtypeStruct((T, _H), jnp.float32),
    )(x, w, yt3, shared_gate_w, shared_up_w, shared_down_w)

    return out.reshape(b, s, hd), logits
